# baseline (speedup 1.0000x reference)
import functools

import jax
import jax.numpy as jnp
from jax import lax
from jax.experimental import pallas as pl
from jax.experimental.pallas import tpu as pltpu
from jax.experimental.pallas import tpu_sc as plsc

D, H, W = 96, 96, 96
M = D * H * W
N = 2048
T = 12288
NV = N + 8

NUM_TILES = 32
PER_TILE = M // NUM_TILES
NBLK = 12
NBUF = 3
BS = PER_TILE // NBLK
ITERS = BS // 16
ROWS = BS // W
OUT_BS = ROWS * 128
OUT_COMP = (M // W) * 128


def _interp_kernel(simp_hbm, disp_hbm, pads_hbm, s_hbm, coords_hbm, out_hbm,
                   simp_v, vals_v, *scr):
    wid = lax.axis_index("s") * 2 + lax.axis_index("c")
    s_bufs = scr[0:NBUF]
    c_bufs = scr[NBUF:2 * NBUF]
    o_bufs = scr[2 * NBUF:3 * NBUF]
    tbl_sem = scr[3 * NBUF]
    in_sems = scr[3 * NBUF + 1:3 * NBUF + 1 + NBUF]
    out_sems = scr[3 * NBUF + 1 + NBUF:3 * NBUF + 1 + 2 * NBUF]

    th0 = pltpu.async_copy(simp_hbm, simp_v, tbl_sem)
    th1 = pltpu.async_copy(disp_hbm, vals_v.at[pl.ds(0, 3 * N)], tbl_sem)
    th2 = pltpu.async_copy(pads_hbm, vals_v.at[pl.ds(3 * N, 24)], tbl_sem)

    def start_in(b, p):
        base = (wid * NBLK + b) * BS
        h0 = pltpu.async_copy(s_hbm.at[pl.ds(base, BS)], s_bufs[p], in_sems[p])
        h1 = pltpu.async_copy(coords_hbm.at[pl.ds(base * 4, BS * 4)],
                              c_bufs[p], in_sems[p])
        return (h0, h1)

    def start_out(b, p):
        row0 = (wid * NBLK + b) * ROWS
        return tuple(
            pltpu.async_copy(o_bufs[p].at[pl.ds(k * OUT_BS, OUT_BS)],
                             out_hbm.at[pl.ds(k * OUT_COMP + row0 * 128, OUT_BS)],
                             out_sems[p])
            for k in range(3))

    pend_in = [start_in(b, b) for b in range(NBUF)]
    pend_out = [None] * NBUF

    zv = jnp.zeros((16,), jnp.float32)
    for o_v in o_bufs:
        for r in range(3 * ROWS):
            o_v[pl.ds(r * 128 + 96, 16)] = zv
            o_v[pl.ds(r * 128 + 112, 16)] = zv

    th0.wait()
    th1.wait()
    th2.wait()

    for b in range(NBLK):
        p = b % NBUF
        for h in pend_in[p]:
            h.wait()
        if pend_out[p] is not None:
            for h in pend_out[p]:
                h.wait()
        s_v, coords_v, out_v = s_bufs[p], c_bufs[p], o_bufs[p]

        @plsc.parallel_loop(0, ITERS, unroll=4)
        def body(i):
            sv = s_v[pl.ds(i * 16, 16)]
            sbase = ((sv >> 7) << 9) + (sv & 127)
            cslot = (i >> 3) * 512 + (i & 7) * 16
            res0 = jnp.zeros((16,), jnp.float32)
            res1 = jnp.zeros((16,), jnp.float32)
            res2 = jnp.zeros((16,), jnp.float32)
            for v in range(4):
                vert = plsc.load_gather(simp_v, [sbase + v * 128])
                wv = coords_v[pl.ds(cslot + v * 128, 16)]
                pad = vert >= N
                res0 += wv * plsc.load_gather(
                    vals_v, [vert + jnp.where(pad, 2 * N, 0)])
                res1 += wv * plsc.load_gather(
                    vals_v, [vert + jnp.where(pad, 2 * N + 8, N)])
                res2 += wv * plsc.load_gather(
                    vals_v, [vert + jnp.where(pad, 2 * N + 16, 2 * N)])
            obase = (i // 6) * 128 + (i % 6) * 16
            out_v[pl.ds(obase, 16)] = res0
            out_v[pl.ds(OUT_BS + obase, 16)] = res1
            out_v[pl.ds(2 * OUT_BS + obase, 16)] = res2

        if b + NBUF < NBLK:
            pend_in[p] = start_in(b + NBUF, p)
        pend_out[p] = start_out(b, p)

    for hs in pend_out:
        for h in hs:
            h.wait()


@jax.jit
def _run(simp_flat, disp_flat, pads_flat, s, coords_flat):
    mesh = plsc.VectorSubcoreMesh(core_axis_name="c", subcore_axis_name="s")
    kern = functools.partial(
        pl.kernel,
        out_type=jax.ShapeDtypeStruct((3 * OUT_COMP,), jnp.float32),
        mesh=mesh,
        compiler_params=pltpu.CompilerParams(needs_layout_passes=False),
        scratch_types=[
            pltpu.VMEM((T * 4,), jnp.int32),
            pltpu.VMEM((NV * 3,), jnp.float32),
            *([pltpu.VMEM((BS,), jnp.int32)] * NBUF),
            *([pltpu.VMEM((BS * 4,), jnp.float32)] * NBUF),
            *([pltpu.VMEM((3 * OUT_BS,), jnp.float32)] * NBUF),
            *([pltpu.SemaphoreType.DMA] * (1 + 2 * NBUF)),
        ],
    )(_interp_kernel)
    return kern(simp_flat, disp_flat, pads_flat, s, coords_flat)


def kernel(kpts, disp, pads, pads_values, simplices, coords, s):
    disp_flat = disp[0].T.reshape(-1)
    pads_flat = pads_values[0].T.reshape(-1)
    simp_flat = simplices.reshape(T // 128, 128, 4).transpose(0, 2, 1).reshape(-1)
    coords_flat = coords.reshape(M // 128, 128, 4).transpose(0, 2, 1).reshape(-1)
    out_flat = _run(simp_flat, disp_flat, pads_flat, s, coords_flat)
    return out_flat.reshape(1, 3, D, H, 128)[..., :W]


# speedup vs baseline: 1.0353x; 1.0353x over previous
import functools

import jax
import jax.numpy as jnp
from jax import lax
from jax.experimental import pallas as pl
from jax.experimental.pallas import tpu as pltpu
from jax.experimental.pallas import tpu_sc as plsc

D, H, W = 96, 96, 96
M = D * H * W
N = 2048
T = 12288
NV = N + 8

NUM_TILES = 32
PER_TILE = M // NUM_TILES
NBLK = 12
NBUF = 3
BS = PER_TILE // NBLK
ITERS = BS // 16
ROWS = BS // W
OUT_BS = ROWS * 128
OUT_COMP = (M // W) * 128


def _interp_kernel(simp_hbm, vals_hbm, s_hbm, coords_hbm, out_hbm,
                   simp_v, vals_v, *scr):
    wid = lax.axis_index("s") * 2 + lax.axis_index("c")
    s_bufs = scr[0:NBUF]
    c_bufs = scr[NBUF:2 * NBUF]
    o_bufs = scr[2 * NBUF:3 * NBUF]
    tbl_sem = scr[3 * NBUF]
    in_sems = scr[3 * NBUF + 1:3 * NBUF + 1 + NBUF]
    out_sems = scr[3 * NBUF + 1 + NBUF:3 * NBUF + 1 + 2 * NBUF]

    th0 = pltpu.async_copy(simp_hbm, simp_v, tbl_sem)
    th1 = pltpu.async_copy(vals_hbm, vals_v, tbl_sem)

    def start_in(b, p):
        base = (wid * NBLK + b) * BS
        h0 = pltpu.async_copy(s_hbm.at[pl.ds(base, BS)], s_bufs[p], in_sems[p])
        h1 = pltpu.async_copy(coords_hbm.at[pl.ds(base * 4, BS * 4)],
                              c_bufs[p], in_sems[p])
        return (h0, h1)

    def start_out(b, p):
        row0 = (wid * NBLK + b) * ROWS
        return tuple(
            pltpu.async_copy(o_bufs[p].at[pl.ds(k * OUT_BS, OUT_BS)],
                             out_hbm.at[pl.ds(k * OUT_COMP + row0 * 128, OUT_BS)],
                             out_sems[p])
            for k in range(3))

    pend_in = [start_in(0, 0), start_in(1, 1), None]
    pend_out = [None] * NBUF

    zv = jnp.zeros((16,), jnp.float32)
    for o_v in o_bufs:
        for r in range(3 * ROWS):
            o_v[pl.ds(r * 128 + 96, 16)] = zv
            o_v[pl.ds(r * 128 + 112, 16)] = zv

    th0.wait()
    th1.wait()

    for b in range(NBLK):
        p = b % NBUF
        if b + 2 < NBLK:
            pend_in[(b + 2) % NBUF] = start_in(b + 2, (b + 2) % NBUF)
        for h in pend_in[p]:
            h.wait()
        if pend_out[p] is not None:
            for h in pend_out[p]:
                h.wait()
        s_v, coords_v, out_v = s_bufs[p], c_bufs[p], o_bufs[p]

        @plsc.parallel_loop(0, ITERS, unroll=4)
        def body(i):
            sv = s_v[pl.ds(i * 16, 16)]
            sbase = ((sv >> 7) << 9) + (sv & 127)
            cslot = (i >> 3) * 512 + (i & 7) * 16
            res0 = jnp.zeros((16,), jnp.float32)
            res1 = jnp.zeros((16,), jnp.float32)
            res2 = jnp.zeros((16,), jnp.float32)
            for v in range(4):
                vert = plsc.load_gather(simp_v, [sbase + v * 128])
                wv = coords_v[pl.ds(cslot + v * 128, 16)]
                res0 += wv * plsc.load_gather(vals_v, [vert])
                res1 += wv * plsc.load_gather(vals_v, [vert + NV])
                res2 += wv * plsc.load_gather(vals_v, [vert + 2 * NV])
            obase = (i // 6) * 128 + (i % 6) * 16
            out_v[pl.ds(obase, 16)] = res0
            out_v[pl.ds(OUT_BS + obase, 16)] = res1
            out_v[pl.ds(2 * OUT_BS + obase, 16)] = res2

        pend_out[p] = start_out(b, p)

    for hs in pend_out:
        for h in hs:
            h.wait()


@jax.jit
def _run(simp_flat, vals_flat, s, coords_flat):
    mesh = plsc.VectorSubcoreMesh(core_axis_name="c", subcore_axis_name="s")
    kern = functools.partial(
        pl.kernel,
        out_type=jax.ShapeDtypeStruct((3 * OUT_COMP,), jnp.float32),
        mesh=mesh,
        compiler_params=pltpu.CompilerParams(needs_layout_passes=False),
        scratch_types=[
            pltpu.VMEM((T * 4,), jnp.int32),
            pltpu.VMEM((NV * 3,), jnp.float32),
            *([pltpu.VMEM((BS,), jnp.int32)] * NBUF),
            *([pltpu.VMEM((BS * 4,), jnp.float32)] * NBUF),
            *([pltpu.VMEM((3 * OUT_BS,), jnp.float32)] * NBUF),
            *([pltpu.SemaphoreType.DMA] * (1 + 2 * NBUF)),
        ],
    )(_interp_kernel)
    return kern(simp_flat, vals_flat, s, coords_flat)


def kernel(kpts, disp, pads, pads_values, simplices, coords, s):
    vals_flat = jnp.concatenate(
        [disp[0].T, pads_values[0].T], axis=1).reshape(-1)
    simp_flat = simplices.reshape(T // 128, 128, 4).transpose(0, 2, 1).reshape(-1)
    coords_flat = coords.reshape(M // 128, 128, 4).transpose(0, 2, 1).reshape(-1)
    out_flat = _run(simp_flat, vals_flat, s, coords_flat)
    return out_flat.reshape(1, 3, D, H, 128)[..., :W]


# speedup vs baseline: 1.1619x; 1.1223x over previous
import functools

import jax
import jax.numpy as jnp
from jax import lax
from jax.experimental import pallas as pl
from jax.experimental.pallas import tpu as pltpu
from jax.experimental.pallas import tpu_sc as plsc

D, H, W = 96, 96, 96
M = D * H * W
N = 2048
T = 12288
NV = N + 8

NUM_TILES = 32
PER_TILE = M // NUM_TILES
NBLK = 8
NBUF = 2
BS = PER_TILE // NBLK
ITERS = BS // 16
ROWS = BS // W
OUT_BS = ROWS * 128
OUT_COMP = (M // W) * 128


def _interp_kernel(sp01_hbm, sp23_hbm, vals_hbm, s_hbm, coords_hbm, out_hbm,
                   sp01_v, sp23_v, vals_v, *scr):
    wid = lax.axis_index("s") * 2 + lax.axis_index("c")
    s_bufs = scr[0:NBUF]
    c_bufs = scr[NBUF:2 * NBUF]
    o_bufs = scr[2 * NBUF:3 * NBUF]
    tbl_sem = scr[3 * NBUF]
    in_sems = scr[3 * NBUF + 1:3 * NBUF + 1 + NBUF]
    out_sems = scr[3 * NBUF + 1 + NBUF:3 * NBUF + 1 + 2 * NBUF]

    th0 = pltpu.async_copy(sp01_hbm, sp01_v, tbl_sem)
    th1 = pltpu.async_copy(sp23_hbm, sp23_v, tbl_sem)
    th2 = pltpu.async_copy(vals_hbm, vals_v, tbl_sem)

    def start_in(b, p):
        base = (wid * NBLK + b) * BS
        h0 = pltpu.async_copy(s_hbm.at[pl.ds(base, BS)], s_bufs[p], in_sems[p])
        h1 = pltpu.async_copy(coords_hbm.at[pl.ds(base * 4, BS * 4)],
                              c_bufs[p], in_sems[p])
        return (h0, h1)

    def start_out(b, p):
        row0 = (wid * NBLK + b) * ROWS
        return tuple(
            pltpu.async_copy(o_bufs[p].at[pl.ds(k * OUT_BS, OUT_BS)],
                             out_hbm.at[pl.ds(k * OUT_COMP + row0 * 128, OUT_BS)],
                             out_sems[p])
            for k in range(3))

    pend_in = [start_in(b, b) for b in range(NBUF)]
    pend_out = [None] * NBUF

    zv = jnp.zeros((16,), jnp.float32)
    for o_v in o_bufs:
        for r in range(3 * ROWS):
            o_v[pl.ds(r * 128 + 96, 16)] = zv
            o_v[pl.ds(r * 128 + 112, 16)] = zv

    th0.wait()
    th1.wait()
    th2.wait()

    for b in range(NBLK):
        p = b % NBUF
        for h in pend_in[p]:
            h.wait()
        if pend_out[p] is not None:
            for h in pend_out[p]:
                h.wait()
        s_v, coords_v, out_v = s_bufs[p], c_bufs[p], o_bufs[p]

        @plsc.parallel_loop(0, ITERS, unroll=4)
        def body(i):
            sv = s_v[pl.ds(i * 16, 16)]
            cslot = (i >> 3) * 512 + (i & 7) * 16
            res0 = jnp.zeros((16,), jnp.float32)
            res1 = jnp.zeros((16,), jnp.float32)
            res2 = jnp.zeros((16,), jnp.float32)
            vp01 = plsc.load_gather(sp01_v, [sv])
            vp23 = plsc.load_gather(sp23_v, [sv])
            verts = (vp01 & 0xFFFF, vp01 >> 16, vp23 & 0xFFFF, vp23 >> 16)
            for v in range(4):
                vert = verts[v]
                wv = coords_v[pl.ds(cslot + v * 128, 16)]
                res0 += wv * plsc.load_gather(vals_v, [vert])
                res1 += wv * plsc.load_gather(vals_v, [vert + NV])
                res2 += wv * plsc.load_gather(vals_v, [vert + 2 * NV])
            obase = (i // 6) * 128 + (i % 6) * 16
            out_v[pl.ds(obase, 16)] = res0
            out_v[pl.ds(OUT_BS + obase, 16)] = res1
            out_v[pl.ds(2 * OUT_BS + obase, 16)] = res2

        if b + NBUF < NBLK:
            pend_in[p] = start_in(b + NBUF, p)
        pend_out[p] = start_out(b, p)

    for hs in pend_out:
        for h in hs:
            h.wait()


@jax.jit
def _run(sp01, sp23, vals_flat, s, coords_flat):
    mesh = plsc.VectorSubcoreMesh(core_axis_name="c", subcore_axis_name="s")
    kern = functools.partial(
        pl.kernel,
        out_type=jax.ShapeDtypeStruct((3 * OUT_COMP,), jnp.float32),
        mesh=mesh,
        compiler_params=pltpu.CompilerParams(needs_layout_passes=False),
        scratch_types=[
            pltpu.VMEM((T,), jnp.int32),
            pltpu.VMEM((T,), jnp.int32),
            pltpu.VMEM((NV * 3,), jnp.float32),
            *([pltpu.VMEM((BS,), jnp.int32)] * NBUF),
            *([pltpu.VMEM((BS * 4,), jnp.float32)] * NBUF),
            *([pltpu.VMEM((3 * OUT_BS,), jnp.float32)] * NBUF),
            *([pltpu.SemaphoreType.DMA] * (1 + 2 * NBUF)),
        ],
    )(_interp_kernel)
    return kern(sp01, sp23, vals_flat, s, coords_flat)


def kernel(kpts, disp, pads, pads_values, simplices, coords, s):
    vals_flat = jnp.concatenate(
        [disp[0].T, pads_values[0].T], axis=1).reshape(-1)
    sp01 = simplices[:, 0] | (simplices[:, 1] << 16)
    sp23 = simplices[:, 2] | (simplices[:, 3] << 16)
    coords_flat = coords.reshape(M // 128, 128, 4).transpose(0, 2, 1).reshape(-1)
    out_flat = _run(sp01, sp23, vals_flat, s, coords_flat)
    return out_flat.reshape(1, 3, D, H, 128)[..., :W]
